# baseline (speedup 1.0000x reference)
import jax
import jax.numpy as jnp
from jax.experimental import pallas as pl
from jax.experimental.pallas import tpu as pltpu

NODE_DIM = 1024
NUM_TYPES = 16
BATCH = 256


def _mix_body(at_ref, z_ref, o_ref):
    o_ref[...] = jax.lax.dot(at_ref[...], z_ref[...],
                             precision=jax.lax.Precision.HIGHEST,
                             preferred_element_type=jnp.float32)


def _mlp_body(az_ref, g_ref, bt_ref, w1_ref, b1_ref, w2_ref, b2_ref, o_ref):
    x = az_ref[0]
    mu = jnp.mean(x, axis=1, keepdims=True)
    m2 = jnp.mean(x * x, axis=1, keepdims=True)
    azn = (x - mu) * jax.lax.rsqrt(m2 - mu * mu + 1e-5) * g_ref[0, :] \
        + bt_ref[0, :]
    azb = azn.astype(jnp.bfloat16)
    h = jnp.dot(azb, w1_ref[0].astype(jnp.bfloat16),
                preferred_element_type=jnp.float32) + b1_ref[0, 0, :]
    h = jnp.where(h > 0, h, jnp.exp(h) - 1.0)
    mix = jnp.dot(h.astype(jnp.bfloat16), w2_ref[0].astype(jnp.bfloat16),
                  preferred_element_type=jnp.float32) + b2_ref[0, 0, :]
    o_ref[0, :, :] = mix + azn


def kernel(z, A, gamma, beta, W1, b1, W2, b2):
    K = NUM_TYPES
    d = NODE_DIM
    B = z.shape[0] // K
    N = B * d
    zt = z.reshape(B, K, d).transpose(1, 0, 2).reshape(K, N)
    at = A.T
    g2 = gamma.reshape(1, d)
    bt2 = beta.reshape(1, d)
    b1r = b1.reshape(K, 1, 2 * d)
    b2r = b2.reshape(K, 1, d)

    nchunk = 4
    az = pl.pallas_call(
        _mix_body,
        grid=(nchunk,),
        in_specs=[
            pl.BlockSpec((K, K), lambda c: (0, 0)),
            pl.BlockSpec((K, N // nchunk), lambda c: (0, c)),
        ],
        out_specs=pl.BlockSpec((K, N // nchunk), lambda c: (0, c)),
        out_shape=jax.ShapeDtypeStruct((K, N), jnp.float32),
        compiler_params=pltpu.CompilerParams(
            dimension_semantics=("arbitrary",),
        ),
    )(at, zt)
    az3 = az.reshape(K, B, d)

    out = pl.pallas_call(
        _mlp_body,
        grid=(K,),
        in_specs=[
            pl.BlockSpec((1, B, d), lambda k: (k, 0, 0)),
            pl.BlockSpec((1, d), lambda k: (0, 0)),
            pl.BlockSpec((1, d), lambda k: (0, 0)),
            pl.BlockSpec((1, d, 2 * d), lambda k: (k, 0, 0)),
            pl.BlockSpec((1, 1, 2 * d), lambda k: (k, 0, 0)),
            pl.BlockSpec((1, 2 * d, d), lambda k: (k, 0, 0)),
            pl.BlockSpec((1, 1, d), lambda k: (k, 0, 0)),
        ],
        out_specs=pl.BlockSpec((1, B, d), lambda k: (k, 0, 0)),
        out_shape=jax.ShapeDtypeStruct((K, B, d), jnp.float32),
        compiler_params=pltpu.CompilerParams(
            dimension_semantics=("arbitrary",),
        ),
    )(az3, g2, bt2, W1, b1r, W2, b2r)
    return out.transpose(1, 0, 2).reshape(B * K, d)


# speedup vs baseline: 1.1610x; 1.1610x over previous
import jax
import jax.numpy as jnp
from jax.experimental import pallas as pl
from jax.experimental.pallas import tpu as pltpu

NODE_DIM = 1024
NUM_TYPES = 16
BATCH = 256
NCHUNK = 2


def _mixer_body(at_ref, z_ref, g_ref, bt_ref, w1_ref, b1_ref, w2_ref, b2_ref,
                o_ref, azn_ref):
    k = pl.program_id(0)
    c = pl.program_id(1)

    @pl.when(c == 0)
    def _mix_and_norm():
        acc = at_ref[k, 0] * z_ref[:, 0, :]
        for j in range(1, NUM_TYPES):
            acc = acc + at_ref[k, j] * z_ref[:, j, :]
        mu = jnp.mean(acc, axis=1, keepdims=True)
        m2 = jnp.mean(acc * acc, axis=1, keepdims=True)
        azn_ref[...] = (acc - mu) * jax.lax.rsqrt(m2 - mu * mu + 1e-5) \
            * g_ref[0, :] + bt_ref[0, :]

    azn = azn_ref[...]
    h = jnp.dot(azn, w1_ref[0], preferred_element_type=jnp.float32) \
        + b1_ref[0, 0, :]
    h = jnp.where(h > 0, h, jnp.exp(h) - 1.0)
    part = jnp.dot(h, w2_ref[0], preferred_element_type=jnp.float32)

    @pl.when(c == 0)
    def _first():
        o_ref[:, k, :] = part + azn + b2_ref[0, 0, :]

    @pl.when(c != 0)
    def _rest():
        o_ref[:, k, :] += part


def kernel(z, A, gamma, beta, W1, b1, W2, b2):
    K = NUM_TYPES
    d = NODE_DIM
    B = z.shape[0] // K
    hc = 2 * d // NCHUNK
    zb = z.reshape(B, K, d)
    at = A.T
    g2 = gamma.reshape(1, d)
    bt2 = beta.reshape(1, d)
    b1r = b1.reshape(K, 1, 2 * d)
    b2r = b2.reshape(K, 1, d)

    out = pl.pallas_call(
        _mixer_body,
        grid=(K, NCHUNK),
        in_specs=[
            pl.BlockSpec(memory_space=pltpu.SMEM),
            pl.BlockSpec((B, K, d), lambda k, c: (0, 0, 0)),
            pl.BlockSpec((1, d), lambda k, c: (0, 0)),
            pl.BlockSpec((1, d), lambda k, c: (0, 0)),
            pl.BlockSpec((1, d, hc), lambda k, c: (k, 0, c)),
            pl.BlockSpec((1, 1, hc), lambda k, c: (k, 0, c)),
            pl.BlockSpec((1, hc, d), lambda k, c: (k, c, 0)),
            pl.BlockSpec((1, 1, d), lambda k, c: (k, 0, 0)),
        ],
        out_specs=pl.BlockSpec((B, K, d), lambda k, c: (0, 0, 0)),
        out_shape=jax.ShapeDtypeStruct((B, K, d), jnp.float32),
        scratch_shapes=[pltpu.VMEM((B, d), jnp.float32)],
        compiler_params=pltpu.CompilerParams(
            dimension_semantics=("arbitrary", "arbitrary"),
        ),
    )(at, zb, g2, bt2, W1, b1r, W2, b2r)
    return out.reshape(B * K, d)
